# Initial kernel scaffold; baseline (speedup 1.0000x reference)
import jax
import jax.numpy as jnp
from jax.experimental import pallas as pl


def kernel(x, router_w1, router_b1, router_w2, router_b2, W1, b1, W2, b2):
    raise NotImplementedError("write your pallas kernel here")


# speedup vs baseline: 1.6417x; 1.6417x over previous
import functools

import jax
import jax.numpy as jnp
from jax.experimental import pallas as pl
from jax.experimental.pallas import tpu as pltpu
from jax.experimental.pallas import tpu_sc as plsc

_T = 2048
_D = 768
_FF = 3072
_E = 8
_K = 2
_BTR = 256
_BT = 256
_P = _K * _T + _E * _BT
_G = _P // _BT
_GW = 128
_GC = 256


def _gelu_exact(v):
    return 0.5 * v * (1.0 + jax.lax.erf(v * (2.0 ** -0.5)))


def _router_body(logits_ref, idx_ref, w_ref, usage_ref):
    g = pl.program_id(0)
    logits = logits_ref[...]
    cols = jax.lax.broadcasted_iota(jnp.int32, (_BTR, _E), 1)
    m1 = jnp.max(logits, axis=-1, keepdims=True)
    a1 = jnp.min(jnp.where(logits == m1, cols, _E), axis=-1, keepdims=True)
    rest = jnp.where(cols == a1, -jnp.inf, logits)
    m2 = jnp.max(rest, axis=-1, keepdims=True)
    a2 = jnp.min(jnp.where(rest == m2, cols, _E), axis=-1, keepdims=True)
    e2 = jnp.exp(m2 - m1)
    w1 = 1.0 / (1.0 + e2)
    w2 = e2 * w1
    idx_ref[...] = jnp.concatenate([a1, a2], axis=1)
    w_ref[...] = jnp.concatenate([w1, w2], axis=1)
    ex = jnp.exp(logits - m1)
    p = ex / jnp.sum(ex, axis=-1, keepdims=True)

    @pl.when(g == 0)
    def _():
        usage_ref[...] = jnp.zeros_like(usage_ref)

    usage_ref[...] += jnp.sum(p, axis=0, keepdims=True)


def _router(logits):
    grid = (_T // _BTR,)
    return pl.pallas_call(
        _router_body,
        grid=grid,
        in_specs=[
            pl.BlockSpec((_BTR, _E), lambda g: (g, 0)),
        ],
        out_specs=[
            pl.BlockSpec((_BTR, _K), lambda g: (g, 0)),
            pl.BlockSpec((_BTR, _K), lambda g: (g, 0)),
            pl.BlockSpec((1, _E), lambda g: (0, 0)),
        ],
        out_shape=[
            jax.ShapeDtypeStruct((_T, _K), jnp.int32),
            jax.ShapeDtypeStruct((_T, _K), jnp.float32),
            jax.ShapeDtypeStruct((1, _E), jnp.float32),
        ],
        compiler_params=pltpu.CompilerParams(
            dimension_semantics=("arbitrary",)),
    )(logits)


def _routing_setup(top2i, top2w):
    e_flat = top2i.reshape(-1)
    oh = (e_flat[:, None] == jnp.arange(_E)[None, :]).astype(jnp.int32)
    counts = jnp.sum(oh, axis=0)
    ranks = jnp.cumsum(oh, axis=0) - oh
    rank = jnp.sum(ranks * oh, axis=1)
    padded = ((counts + _BT - 1) // _BT) * _BT
    ends = jnp.cumsum(padded)
    starts = ends - padded
    pos = starts[e_flat] + rank
    src_tok = jnp.zeros((_P,), jnp.int32).at[pos].set(
        jnp.arange(_K * _T, dtype=jnp.int32) // _K)
    w_sorted = jnp.zeros((_P,), jnp.float32).at[pos].set(top2w.reshape(-1))
    gstart = jnp.arange(_G, dtype=jnp.int32) * _BT
    block_expert = jnp.clip(
        jnp.searchsorted(ends, gstart, side="right"), 0, _E - 1
    ).astype(jnp.int32)
    block_active = (gstart < ends[-1]).astype(jnp.int32)
    return src_tok, w_sorted, pos.astype(jnp.int32), block_expert, block_active


def _gather_rows(data, idx):
    m = idx.shape[0]
    d = data.shape[1]
    s = d // _GC
    data_c = data.reshape(-1, _GC)
    idx_c = (idx[:, None] * s
             + jnp.arange(s, dtype=jnp.int32)[None, :]).reshape(1, m * s)
    mesh = plsc.VectorSubcoreMesh(core_axis_name="core",
                                  subcore_axis_name="subcore")

    @pl.kernel(out_type=jax.ShapeDtypeStruct((m * s, _GC), data.dtype),
               mesh=mesh)
    def k(x_hbm, i_hbm, o_hbm):
        def body(i_vmem, o_vmem):
            pltpu.sync_copy(x_hbm.at[i_vmem.at[0]], o_vmem)

        pltpu.emit_pipeline(
            body,
            grid=(m * s // _GW,),
            in_specs=[pl.BlockSpec((1, _GW), index_map=lambda i: (0, i))],
            out_specs=[pl.BlockSpec((_GW, _GC), index_map=lambda i: (i, 0))],
            core_axis_name=("core", "subcore"),
            dimension_semantics=(pltpu.PARALLEL,),
        )(i_hbm, o_hbm)

    return k(data_c, idx_c).reshape(m, d)


def _gmm_body(be_ref, act_ref, x_ref, w1_ref, b1_ref, w2_ref, b2_ref,
              ws_ref, o_ref):
    g = pl.program_id(0)

    @pl.when(act_ref[g] == 1)
    def _():
        x = x_ref[...]
        h = jnp.dot(x, w1_ref[0], preferred_element_type=jnp.float32)
        h = _gelu_exact(h + b1_ref[0])
        y = jnp.dot(h, w2_ref[0], preferred_element_type=jnp.float32)
        o_ref[...] = (y + b2_ref[0]) * ws_ref[...]


def _gmm(x_sorted, w_sorted, block_expert, block_active, W1, b1, W2, b2):
    grid_spec = pltpu.PrefetchScalarGridSpec(
        num_scalar_prefetch=2,
        grid=(_G,),
        in_specs=[
            pl.BlockSpec((_BT, _D), lambda g, be, act: (g, 0)),
            pl.BlockSpec((1, _D, _FF), lambda g, be, act: (be[g], 0, 0)),
            pl.BlockSpec((1, 1, _FF), lambda g, be, act: (be[g], 0, 0)),
            pl.BlockSpec((1, _FF, _D), lambda g, be, act: (be[g], 0, 0)),
            pl.BlockSpec((1, 1, _D), lambda g, be, act: (be[g], 0, 0)),
            pl.BlockSpec((_BT, 1), lambda g, be, act: (g, 0)),
        ],
        out_specs=pl.BlockSpec((_BT, _D), lambda g, be, act: (g, 0)),
    )
    return pl.pallas_call(
        _gmm_body,
        grid_spec=grid_spec,
        out_shape=jax.ShapeDtypeStruct((_P, _D), jnp.float32),
        compiler_params=pltpu.CompilerParams(
            dimension_semantics=("arbitrary",)),
    )(block_expert, block_active, x_sorted,
      W1, b1.reshape(_E, 1, _FF), W2, b2.reshape(_E, 1, _D),
      w_sorted.reshape(_P, 1))


def _combine_body(pair_ref, o_ref):
    o_ref[...] = pair_ref[:, 0, :] + pair_ref[:, 1, :]


def _combine(pairs):
    grid = (_T // _BTR,)
    return pl.pallas_call(
        _combine_body,
        grid=grid,
        in_specs=[pl.BlockSpec((_BTR, _K, _D), lambda g: (g, 0, 0))],
        out_specs=pl.BlockSpec((_BTR, _D), lambda g: (g, 0)),
        out_shape=jax.ShapeDtypeStruct((_T, _D), jnp.float32),
        compiler_params=pltpu.CompilerParams(
            dimension_semantics=("arbitrary",)),
    )(pairs)


def kernel(x, router_w1, router_b1, router_w2, router_b2, W1, b1, W2, b2):
    batch, seq, d_model = x.shape
    x_flat = x.reshape(-1, d_model)

    h = jax.nn.gelu(x_flat @ router_w1 + router_b1, approximate=False)
    logits = h @ router_w2 + router_b2

    top2i, top2w, usage_sum = _router(logits)
    usage = usage_sum[0] / jnp.float32(_T)
    lb_loss = 0.01 * jnp.sum((usage - jnp.mean(usage)) ** 2) / (_E - 1)

    src_tok, w_sorted, pos, block_expert, block_active = _routing_setup(
        top2i, top2w)

    x_sorted = _gather_rows(x_flat, src_tok)
    y_scaled = _gmm(x_sorted, w_sorted, block_expert, block_active,
                    W1, b1, W2, b2)
    pairs = _gather_rows(y_scaled, pos).reshape(_T, _K, _D)
    out = _combine(pairs)
    return out.reshape(batch, seq, d_model), lb_loss


# speedup vs baseline: 1.6426x; 1.0005x over previous
import functools

import jax
import jax.numpy as jnp
from jax.experimental import pallas as pl
from jax.experimental.pallas import tpu as pltpu
from jax.experimental.pallas import tpu_sc as plsc

_T = 2048
_D = 768
_FF = 3072
_E = 8
_K = 2
_BTR = 256
_BT = 256
_P = _K * _T + _E * _BT
_G = _P // _BT
_GW = 128
_GC = 256


def _gelu_exact(v):
    return 0.5 * v * (1.0 + jax.lax.erf(v * (2.0 ** -0.5)))


def _router_body(logits_ref, idx_ref, w_ref, usage_ref):
    g = pl.program_id(0)
    logits = logits_ref[...]
    cols = jax.lax.broadcasted_iota(jnp.int32, (_BTR, _E), 1)
    m1 = jnp.max(logits, axis=-1, keepdims=True)
    a1 = jnp.min(jnp.where(logits == m1, cols, _E), axis=-1, keepdims=True)
    rest = jnp.where(cols == a1, -jnp.inf, logits)
    m2 = jnp.max(rest, axis=-1, keepdims=True)
    a2 = jnp.min(jnp.where(rest == m2, cols, _E), axis=-1, keepdims=True)
    e2 = jnp.exp(m2 - m1)
    w1 = 1.0 / (1.0 + e2)
    w2 = e2 * w1
    idx_ref[...] = jnp.concatenate([a1, a2], axis=1)
    w_ref[...] = jnp.concatenate([w1, w2], axis=1)
    ex = jnp.exp(logits - m1)
    p = ex / jnp.sum(ex, axis=-1, keepdims=True)

    @pl.when(g == 0)
    def _():
        usage_ref[...] = jnp.zeros_like(usage_ref)

    usage_ref[...] += jnp.sum(p, axis=0, keepdims=True)


def _router(logits):
    grid = (_T // _BTR,)
    return pl.pallas_call(
        _router_body,
        grid=grid,
        in_specs=[
            pl.BlockSpec((_BTR, _E), lambda g: (g, 0)),
        ],
        out_specs=[
            pl.BlockSpec((_BTR, _K), lambda g: (g, 0)),
            pl.BlockSpec((_BTR, _K), lambda g: (g, 0)),
            pl.BlockSpec((1, _E), lambda g: (0, 0)),
        ],
        out_shape=[
            jax.ShapeDtypeStruct((_T, _K), jnp.int32),
            jax.ShapeDtypeStruct((_T, _K), jnp.float32),
            jax.ShapeDtypeStruct((1, _E), jnp.float32),
        ],
        compiler_params=pltpu.CompilerParams(
            dimension_semantics=("arbitrary",)),
    )(logits)


def _routing_setup(top2i, top2w):
    e_flat = top2i.reshape(-1)
    oh = (e_flat[:, None] == jnp.arange(_E)[None, :]).astype(jnp.int32)
    counts = jnp.sum(oh, axis=0)
    ranks = jnp.cumsum(oh, axis=0) - oh
    rank = jnp.sum(ranks * oh, axis=1)
    padded = ((counts + _BT - 1) // _BT) * _BT
    ends = jnp.cumsum(padded)
    starts = ends - padded
    pos = starts[e_flat] + rank
    src_tok = jnp.zeros((_P,), jnp.int32).at[pos].set(
        jnp.arange(_K * _T, dtype=jnp.int32) // _K)
    w_sorted = jnp.zeros((_P,), jnp.float32).at[pos].set(top2w.reshape(-1))
    gstart = jnp.arange(_G, dtype=jnp.int32) * _BT
    block_expert = jnp.clip(
        jnp.searchsorted(ends, gstart, side="right"), 0, _E - 1
    ).astype(jnp.int32)
    block_active = (gstart < ends[-1]).astype(jnp.int32)
    return src_tok, w_sorted, pos.astype(jnp.int32), block_expert, block_active


def _gather_rows(data, idx):
    m = idx.shape[0]
    d = data.shape[1]
    s = d // _GC
    data_c = data.reshape(-1, _GC)
    idx_c = (idx[:, None] * s
             + jnp.arange(s, dtype=jnp.int32)[None, :]).reshape(1, m * s)
    mesh = plsc.VectorSubcoreMesh(core_axis_name="core",
                                  subcore_axis_name="subcore")

    @pl.kernel(out_type=jax.ShapeDtypeStruct((m * s, _GC), data.dtype),
               mesh=mesh)
    def k(x_hbm, i_hbm, o_hbm):
        def body(i_vmem, o_vmem):
            pltpu.sync_copy(x_hbm.at[i_vmem.at[0]], o_vmem)

        pltpu.emit_pipeline(
            body,
            grid=(m * s // _GW,),
            in_specs=[pl.BlockSpec((1, _GW), index_map=lambda i: (0, i))],
            out_specs=[pl.BlockSpec((_GW, _GC), index_map=lambda i: (i, 0))],
            core_axis_name=("core", "subcore"),
            dimension_semantics=(pltpu.PARALLEL,),
        )(i_hbm, o_hbm)

    return k(data_c, idx_c).reshape(m, d)


def _gmm_body(be_ref, act_ref, x_ref, w1_ref, b1_ref, w2_ref, b2_ref,
              ws_ref, o_ref):
    g = pl.program_id(0)

    @pl.when(act_ref[g] == 1)
    def _():
        x = x_ref[...]
        h = jnp.dot(x, w1_ref[0], preferred_element_type=jnp.float32,
                    precision=jax.lax.Precision.DEFAULT)
        h = _gelu_exact(h + b1_ref[0])
        y = jnp.dot(h, w2_ref[0], preferred_element_type=jnp.float32,
                    precision=jax.lax.Precision.DEFAULT)
        o_ref[...] = (y + b2_ref[0]) * ws_ref[...]


def _gmm(x_sorted, w_sorted, block_expert, block_active, W1, b1, W2, b2):
    grid_spec = pltpu.PrefetchScalarGridSpec(
        num_scalar_prefetch=2,
        grid=(_G,),
        in_specs=[
            pl.BlockSpec((_BT, _D), lambda g, be, act: (g, 0)),
            pl.BlockSpec((1, _D, _FF), lambda g, be, act: (be[g], 0, 0)),
            pl.BlockSpec((1, 1, _FF), lambda g, be, act: (be[g], 0, 0)),
            pl.BlockSpec((1, _FF, _D), lambda g, be, act: (be[g], 0, 0)),
            pl.BlockSpec((1, 1, _D), lambda g, be, act: (be[g], 0, 0)),
            pl.BlockSpec((_BT, 1), lambda g, be, act: (g, 0)),
        ],
        out_specs=pl.BlockSpec((_BT, _D), lambda g, be, act: (g, 0)),
    )
    return pl.pallas_call(
        _gmm_body,
        grid_spec=grid_spec,
        out_shape=jax.ShapeDtypeStruct((_P, _D), jnp.float32),
        compiler_params=pltpu.CompilerParams(
            dimension_semantics=("arbitrary",)),
    )(block_expert, block_active, x_sorted,
      W1, b1.reshape(_E, 1, _FF), W2, b2.reshape(_E, 1, _D),
      w_sorted.reshape(_P, 1))


def _combine_body(pair_ref, o_ref):
    o_ref[...] = pair_ref[:, 0, :] + pair_ref[:, 1, :]


def _combine(pairs):
    grid = (_T // _BTR,)
    return pl.pallas_call(
        _combine_body,
        grid=grid,
        in_specs=[pl.BlockSpec((_BTR, _K, _D), lambda g: (g, 0, 0))],
        out_specs=pl.BlockSpec((_BTR, _D), lambda g: (g, 0)),
        out_shape=jax.ShapeDtypeStruct((_T, _D), jnp.float32),
        compiler_params=pltpu.CompilerParams(
            dimension_semantics=("arbitrary",)),
    )(pairs)


def kernel(x, router_w1, router_b1, router_w2, router_b2, W1, b1, W2, b2):
    batch, seq, d_model = x.shape
    x_flat = x.reshape(-1, d_model)

    h = jax.nn.gelu(x_flat @ router_w1 + router_b1, approximate=False)
    logits = h @ router_w2 + router_b2

    top2i, top2w, usage_sum = _router(logits)
    usage = usage_sum[0] / jnp.float32(_T)
    lb_loss = 0.01 * jnp.sum((usage - jnp.mean(usage)) ** 2) / (_E - 1)

    src_tok, w_sorted, pos, block_expert, block_active = _routing_setup(
        top2i, top2w)

    x_sorted = _gather_rows(x_flat, src_tok)
    y_scaled = _gmm(x_sorted, w_sorted, block_expert, block_active,
                    W1, b1, W2, b2)
    pairs = _gather_rows(y_scaled, pos).reshape(_T, _K, _D)
    out = _combine(pairs)
    return out.reshape(batch, seq, d_model), lb_loss
